# baseline (speedup 1.0000x reference)
import jax
import jax.numpy as jnp
from jax import lax
from jax.experimental import pallas as pl
from jax.experimental.pallas import tpu as pltpu
from jax.experimental.pallas import tpu_sc as plsc

NUM_EMB = 1000000
DIM = 32
SEQ = 50
NTOK = 16384
B = NTOK * SEQ
NC, NS = 2, 16
NW = NC * NS
PER_W = B // NW
CW = 128
NCELL = PER_W // CW
TPC = NTOK // CW


def _transpose_cell(rows_q, trans_q, row_idx):
    def col(d, carry):
        col_idx = jnp.full((16,), d, dtype=jnp.int32)
        for ch in range(CW // 16):
            v = plsc.load_gather(rows_q, [row_idx[ch], col_idx])
            trans_q[d, pl.ds(ch * 16, 16)] = v
        return carry

    lax.fori_loop(0, DIM, col, 0)


def _transpose_cell_unrolled(rows_q, trans_q, row_idx):
    for d in range(DIM):
        col_idx = jnp.full((16,), d, dtype=jnp.int32)
        for ch in range(CW // 16):
            v = plsc.load_gather(rows_q, [row_idx[ch], col_idx])
            trans_q[d, pl.ds(ch * 16, 16)] = v


def _body(idx_hbm, table_hbm, out_hbm, idx_v, rows0, rows1, trans0, trans1,
          gsem, wsem):
    w = lax.axis_index("s") * NC + lax.axis_index("c")
    pltpu.sync_copy(idx_hbm.at[w], idx_v)
    row_idx = [jnp.arange(16, dtype=jnp.int32) + ch * 16
               for ch in range(CW // 16)]
    rows = (rows0, rows1)
    trans = (trans0, trans1)

    def fire_gather(n, rows_q):
        pltpu.async_copy(table_hbm.at[idx_v.at[n]], rows_q, gsem)

    def drain_gather(rows_q):
        pltpu.make_async_copy(table_hbm.at[pl.ds(0, CW)], rows_q, gsem).wait()

    def fire_writes(n, trans_q):
        c = w * NCELL + n
        s = c // TPC
        tc = lax.rem(c, TPC)
        for tr in range(DIM // 8):
            pltpu.async_copy(trans_q.at[pl.ds(tr * 8, 8)],
                             out_hbm.at[s, tr, tc], wsem)

    def drain_writes(trans_q):
        for tr in range(DIM // 8):
            pltpu.make_async_copy(out_hbm.at[0, 0, 0],
                                  trans_q.at[pl.ds(tr * 8, 8)], wsem).wait()

    def cell(n, q, first, last, unrolled=False):
        if not first:
            drain_writes(trans[q])
        drain_gather(rows[q])
        if unrolled:
            _transpose_cell_unrolled(rows[q], trans[q], row_idx)
        else:
            _transpose_cell(rows[q], trans[q], row_idx)
        fire_writes(n, trans[q])
        if not last:
            fire_gather(n + 2, rows[q])

    fire_gather(0, rows0)
    fire_gather(1, rows1)
    cell(0, 0, first=True, last=False)
    cell(1, 1, first=True, last=False)

    def pair(p, carry):
        cell(2 * p, 0, first=False, last=False, unrolled=True)
        cell(2 * p + 1, 1, first=False, last=False, unrolled=True)
        return carry

    lax.fori_loop(1, NCELL // 2 - 1, pair, 0)

    cell(NCELL - 2, 0, first=False, last=True)
    cell(NCELL - 1, 1, first=False, last=True)
    drain_writes(trans0)
    drain_writes(trans1)


@jax.jit
def _gather(idx, table):
    mesh = plsc.VectorSubcoreMesh(core_axis_name="c", subcore_axis_name="s")
    return pl.kernel(
        _body,
        out_type=jax.ShapeDtypeStruct((SEQ, DIM // 8, TPC, 8, CW),
                                      jnp.float32),
        mesh=mesh,
        scratch_types=[
            pltpu.VMEM((NCELL, CW), jnp.int32),
            pltpu.VMEM((CW, DIM), jnp.float32),
            pltpu.VMEM((CW, DIM), jnp.float32),
            pltpu.VMEM((DIM, CW), jnp.float32),
            pltpu.VMEM((DIM, CW), jnp.float32),
            pltpu.SemaphoreType.DMA,
            pltpu.SemaphoreType.DMA,
        ],
        compiler_params=pltpu.CompilerParams(use_tc_tiling_on_sc=False,
                                             needs_layout_passes=False),
    )(idx, table)


def kernel(token_ids, embedding):
    idx = token_ids.T.reshape(NW, NCELL, CW).astype(jnp.int32)
    out5 = _gather(idx, embedding)
    return out5.transpose(2, 4, 0, 1, 3).reshape(NTOK, SEQ, DIM)


# speedup vs baseline: 1.2043x; 1.2043x over previous
import jax
import jax.numpy as jnp
from jax import lax
from jax.experimental import pallas as pl
from jax.experimental.pallas import tpu as pltpu
from jax.experimental.pallas import tpu_sc as plsc

NUM_EMB = 1000000
DIM = 32
SEQ = 50
NTOK = 16384
B = NTOK * SEQ
NC, NS = 2, 16
NW = NC * NS
PER_W = B // NW
CW = 128
NCELL = PER_W // CW
TPC = NTOK // CW


def _transpose_cell(rows_q, trans_q, iota_lo, iota_hi):
    def tok(j4, carry):
        for u in range(4):
            j = j4 * 4 + u
            cj = jnp.full((16,), j, dtype=jnp.int32)
            v0 = rows_q[j, pl.ds(0, 16)]
            v1 = rows_q[j, pl.ds(16, 16)]
            plsc.store_scatter(trans_q, [iota_lo, cj], v0)
            plsc.store_scatter(trans_q, [iota_hi, cj], v1)
        return carry

    lax.fori_loop(0, CW // 4, tok, 0)


def _body(idx_hbm, table_hbm, out_hbm, idx_v, rows0, rows1, trans0, trans1,
          gsem, wsem):
    w = lax.axis_index("s") * NC + lax.axis_index("c")
    pltpu.sync_copy(idx_hbm.at[w], idx_v)
    iota_lo = jnp.arange(16, dtype=jnp.int32)
    iota_hi = iota_lo + 16
    rows = (rows0, rows1)
    trans = (trans0, trans1)

    def fire_gather(n, rows_q):
        pltpu.async_copy(table_hbm.at[idx_v.at[n]], rows_q, gsem)

    def drain_gather(rows_q):
        pltpu.make_async_copy(table_hbm.at[pl.ds(0, CW)], rows_q, gsem).wait()

    def fire_writes(n, trans_q):
        c = w * NCELL + n
        s = c // TPC
        tc = lax.rem(c, TPC)
        for tr in range(DIM // 8):
            pltpu.async_copy(trans_q.at[pl.ds(tr * 8, 8)],
                             out_hbm.at[s, tr, tc], wsem)

    def drain_writes(trans_q):
        for tr in range(DIM // 8):
            pltpu.make_async_copy(out_hbm.at[0, 0, 0],
                                  trans_q.at[pl.ds(tr * 8, 8)], wsem).wait()

    def cell(n, q, first, last):
        if not first:
            drain_writes(trans[q])
        drain_gather(rows[q])
        _transpose_cell(rows[q], trans[q], iota_lo, iota_hi)
        fire_writes(n, trans[q])
        if not last:
            fire_gather(n + 2, rows[q])

    fire_gather(0, rows0)
    fire_gather(1, rows1)
    cell(0, 0, first=True, last=False)
    cell(1, 1, first=True, last=False)

    def pair(p, carry):
        cell(2 * p, 0, first=False, last=False)
        cell(2 * p + 1, 1, first=False, last=False)
        return carry

    lax.fori_loop(1, NCELL // 2 - 1, pair, 0)

    cell(NCELL - 2, 0, first=False, last=True)
    cell(NCELL - 1, 1, first=False, last=True)
    drain_writes(trans0)
    drain_writes(trans1)


@jax.jit
def _gather(idx, table):
    mesh = plsc.VectorSubcoreMesh(core_axis_name="c", subcore_axis_name="s")
    return pl.kernel(
        _body,
        out_type=jax.ShapeDtypeStruct((SEQ, DIM // 8, TPC, 8, CW),
                                      jnp.float32),
        mesh=mesh,
        scratch_types=[
            pltpu.VMEM((NCELL, CW), jnp.int32),
            pltpu.VMEM((CW, DIM), jnp.float32),
            pltpu.VMEM((CW, DIM), jnp.float32),
            pltpu.VMEM((DIM, CW), jnp.float32),
            pltpu.VMEM((DIM, CW), jnp.float32),
            pltpu.SemaphoreType.DMA,
            pltpu.SemaphoreType.DMA,
        ],
        compiler_params=pltpu.CompilerParams(use_tc_tiling_on_sc=False,
                                             needs_layout_passes=False),
    )(idx, table)


def kernel(token_ids, embedding):
    idx = token_ids.T.reshape(NW, NCELL, CW).astype(jnp.int32)
    out5 = _gather(idx, embedding)
    return out5.transpose(2, 4, 0, 1, 3).reshape(NTOK, SEQ, DIM)
